# Initial kernel scaffold; baseline (speedup 1.0000x reference)
import jax
import jax.numpy as jnp
from jax.experimental import pallas as pl


def kernel(numerical, cat_indices, emb_tables, W1, b1, W2, b2):
    raise NotImplementedError("write your pallas kernel here")


# speedup vs baseline: 2.2055x; 2.2055x over previous
import functools

import jax
import jax.numpy as jnp
from jax import lax
from jax.experimental import pallas as pl
from jax.experimental.pallas import tpu as pltpu
from jax.experimental.pallas import tpu_sc as plsc

B = 16384
NUM = 13
NCAT = 26
VOCAB = 100000
EDIM = 16
HID = 16
NOUT = 1

NW = 32
ROWS_PER_W = B // NW
GROUP_ROWS = 64
GROUPS = ROWS_PER_W // GROUP_ROWS
IDX_PER_GROUP = GROUP_ROWS * NCAT
CHUNK = 128
CHUNKS = IDX_PER_GROUP // CHUNK
IDX_ROWS = B * NCAT // CHUNK
IDX_ROWS_PER_W = IDX_PER_GROUP * GROUPS // CHUNK


def _tc_prep_body(num_ref, w1_ref, b1_ref, w2_ref, b2_ref, base_ref, wtab_ref):
    w = jnp.dot(w1_ref[...], w2_ref[...], preferred_element_type=jnp.float32)
    c = jnp.dot(b1_ref[...], w2_ref[...], preferred_element_type=jnp.float32) + b2_ref[...]
    base = jnp.dot(num_ref[...], w[:NUM, :], preferred_element_type=jnp.float32) + c
    base_ref[...] = base
    wtab_ref[...] = w[NUM:, 0].reshape(NCAT, EDIM)


def _tc_prep(numerical, W1, b1, W2, b2):
    return pl.pallas_call(
        _tc_prep_body,
        out_shape=[
            jax.ShapeDtypeStruct((B, 1), jnp.float32),
            jax.ShapeDtypeStruct((NCAT, EDIM), jnp.float32),
        ],
    )(numerical, W1, b1, W2, b2)


def _tree_sum(terms):
    ts = list(terms)
    while len(ts) > 1:
        nxt = [ts[i] + ts[i + 1] for i in range(0, len(ts) - 1, 2)]
        if len(ts) % 2:
            nxt.append(ts[-1])
        ts = nxt
    return ts[0]


def _sc_body(table_ref, idx_ref, wtab_ref, out_ref, idx_v, gbuf, wbuf, obuf, gsem):
    wid = lax.axis_index("s") * 2 + lax.axis_index("c")
    row0 = wid * ROWS_PER_W

    pltpu.sync_copy(wtab_ref, wbuf)
    pltpu.sync_copy(idx_ref.at[pl.ds(wid * IDX_ROWS_PER_W, IDX_ROWS_PER_W)], idx_v)

    wv = [wbuf[j] for j in range(NCAT)]

    def group_body(g, _):
        cps = [
            pltpu.async_copy(
                table_ref.at[idx_v.at[g * CHUNKS + c]],
                gbuf.at[pl.ds(c * CHUNK, CHUNK)],
                gsem,
            )
            for c in range(CHUNKS)
        ]
        for cp in cps:
            cp.wait()

        def row_body(r, _):
            acc = _tree_sum([gbuf[r * NCAT + j] * wv[j] for j in range(NCAT)])
            obuf[g * GROUP_ROWS + r] = acc
            return 0

        lax.fori_loop(0, GROUP_ROWS, row_body, 0)
        return 0

    lax.fori_loop(0, GROUPS, group_body, 0)
    pltpu.sync_copy(obuf, out_ref.at[pl.ds(row0, ROWS_PER_W)])


def _sc_lookup(flat_table, flat_idx, wtab):
    mesh = plsc.VectorSubcoreMesh(core_axis_name="c", subcore_axis_name="s")
    kfn = pl.kernel(
        _sc_body,
        out_type=jax.ShapeDtypeStruct((B, EDIM), jnp.float32),
        mesh=mesh,
        compiler_params=pltpu.CompilerParams(use_tc_tiling_on_sc=False),
        scratch_types=[
            pltpu.VMEM((IDX_ROWS_PER_W, CHUNK), jnp.int32),
            pltpu.VMEM((IDX_PER_GROUP, EDIM), jnp.float32),
            pltpu.VMEM((NCAT, EDIM), jnp.float32),
            pltpu.VMEM((ROWS_PER_W, EDIM), jnp.float32),
            pltpu.SemaphoreType.DMA,
        ],
    )
    return kfn(flat_table, flat_idx, wtab)


def _tc_finish_body(part_ref, base_ref, out_ref):
    out_ref[...] = (
        jnp.sum(part_ref[...], axis=1, keepdims=True) + base_ref[...]
    )


def _tc_finish(partials, base2d):
    return pl.pallas_call(
        _tc_finish_body,
        out_shape=jax.ShapeDtypeStruct((B, NOUT), jnp.float32),
    )(partials, base2d)


@jax.jit
def kernel(numerical, cat_indices, emb_tables, W1, b1, W2, b2):
    base2d, wtab = _tc_prep(
        numerical, W1, b1.reshape(1, HID), W2, b2.reshape(1, NOUT)
    )
    flat_table = emb_tables.reshape(NCAT * VOCAB, EDIM)
    offs = (jnp.arange(NCAT, dtype=jnp.int32) * VOCAB)[None, :]
    flat_idx = (cat_indices.astype(jnp.int32) + offs).reshape(IDX_ROWS, CHUNK)
    partials = _sc_lookup(flat_table, flat_idx, wtab)
    return _tc_finish(partials, base2d)


# speedup vs baseline: 2.2087x; 1.0014x over previous
import functools

import jax
import jax.numpy as jnp
from jax import lax
from jax.experimental import pallas as pl
from jax.experimental.pallas import tpu as pltpu
from jax.experimental.pallas import tpu_sc as plsc

B = 16384
NUM = 13
NCAT = 26
VOCAB = 100000
EDIM = 16
HID = 16
NOUT = 1

NW = 32
ROWS_PER_W = B // NW
GROUP_ROWS = 64
GROUPS = ROWS_PER_W // GROUP_ROWS
IDX_PER_GROUP = GROUP_ROWS * NCAT
CHUNK = 128
CHUNKS = IDX_PER_GROUP // CHUNK
IDX_ROWS = B * NCAT // CHUNK
IDX_ROWS_PER_W = IDX_PER_GROUP * GROUPS // CHUNK


def _tc_prep_body(num_ref, w1_ref, b1_ref, w2_ref, b2_ref, base_ref, wtab_ref):
    w = jnp.dot(w1_ref[...], w2_ref[...], preferred_element_type=jnp.float32)
    c = jnp.dot(b1_ref[...], w2_ref[...], preferred_element_type=jnp.float32) + b2_ref[...]
    base = jnp.dot(num_ref[...], w[:NUM, :], preferred_element_type=jnp.float32) + c
    base_ref[...] = base
    wtab_ref[...] = w[NUM:, 0].reshape(NCAT, EDIM)


def _tc_prep(numerical, W1, b1, W2, b2):
    return pl.pallas_call(
        _tc_prep_body,
        out_shape=[
            jax.ShapeDtypeStruct((B, 1), jnp.float32),
            jax.ShapeDtypeStruct((NCAT, EDIM), jnp.float32),
        ],
    )(numerical, W1, b1, W2, b2)


def _tree_sum(terms):
    ts = list(terms)
    while len(ts) > 1:
        nxt = [ts[i] + ts[i + 1] for i in range(0, len(ts) - 1, 2)]
        if len(ts) % 2:
            nxt.append(ts[-1])
        ts = nxt
    return ts[0]


def _sc_body(table_ref, idx_ref, wtab_ref, out_ref, idx_v, gbuf, wbuf, obuf, gsem):
    wid = lax.axis_index("s") * 2 + lax.axis_index("c")
    row0 = wid * ROWS_PER_W

    pltpu.sync_copy(wtab_ref, wbuf)
    pltpu.sync_copy(idx_ref.at[wid], idx_v)

    wv = [wbuf[j] for j in range(NCAT)]

    def group_body(g, _):
        cps = [
            pltpu.async_copy(
                table_ref.at[j].at[idx_v.at[j, pl.ds(g * GROUP_ROWS, GROUP_ROWS)]],
                gbuf.at[pl.ds(j * GROUP_ROWS, GROUP_ROWS)],
                gsem,
            )
            for j in range(NCAT)
        ]
        for cp in cps:
            cp.wait()

        def row_body(r, _):
            acc = _tree_sum(
                [gbuf[j * GROUP_ROWS + r] * wv[j] for j in range(NCAT)]
            )
            obuf[g * GROUP_ROWS + r] = acc
            return 0

        lax.fori_loop(0, GROUP_ROWS, row_body, 0)
        return 0

    lax.fori_loop(0, GROUPS, group_body, 0)
    pltpu.sync_copy(obuf, out_ref.at[pl.ds(row0, ROWS_PER_W)])


def _sc_lookup(emb_tables, idx3, wtab):
    mesh = plsc.VectorSubcoreMesh(core_axis_name="c", subcore_axis_name="s")
    kfn = pl.kernel(
        _sc_body,
        out_type=jax.ShapeDtypeStruct((B, EDIM), jnp.float32),
        mesh=mesh,
        compiler_params=pltpu.CompilerParams(use_tc_tiling_on_sc=False),
        scratch_types=[
            pltpu.VMEM((NCAT, ROWS_PER_W), jnp.int32),
            pltpu.VMEM((IDX_PER_GROUP, EDIM), jnp.float32),
            pltpu.VMEM((NCAT, EDIM), jnp.float32),
            pltpu.VMEM((ROWS_PER_W, EDIM), jnp.float32),
            pltpu.SemaphoreType.DMA,
        ],
    )
    return kfn(emb_tables, idx3, wtab)


def _tc_finish_body(part_ref, base_ref, out_ref):
    out_ref[...] = (
        jnp.sum(part_ref[...], axis=1, keepdims=True) + base_ref[...]
    )


def _tc_finish(partials, base2d):
    return pl.pallas_call(
        _tc_finish_body,
        out_shape=jax.ShapeDtypeStruct((B, NOUT), jnp.float32),
    )(partials, base2d)


@jax.jit
def kernel(numerical, cat_indices, emb_tables, W1, b1, W2, b2):
    base2d, wtab = _tc_prep(
        numerical, W1, b1.reshape(1, HID), W2, b2.reshape(1, NOUT)
    )
    idx3 = (
        cat_indices.astype(jnp.int32)
        .reshape(NW, ROWS_PER_W, NCAT)
        .transpose(0, 2, 1)
    )
    partials = _sc_lookup(emb_tables, idx3, wtab)
    return _tc_finish(partials, base2d)


# speedup vs baseline: 21.3819x; 9.6807x over previous
import jax
import jax.numpy as jnp
from jax import lax
from jax.experimental import pallas as pl
from jax.experimental.pallas import tpu as pltpu
from jax.experimental.pallas import tpu_sc as plsc

B = 16384
NUM = 13
NCAT = 26
VOCAB = 100000
EDIM = 16
HID = 16
NOUT = 1

VP = 100352
NW = 32
ROWS_PER_W = B // NW
CHUNK = 128
CHUNKS = ROWS_PER_W // CHUNK


def _tc_prep_body(num_ref, w1_ref, b1_ref, w2_ref, b2_ref, base_ref, wtab_ref):
    w = jnp.dot(w1_ref[...], w2_ref[...], preferred_element_type=jnp.float32)
    c = jnp.dot(b1_ref[...], w2_ref[...], preferred_element_type=jnp.float32) + b2_ref[...]
    base = jnp.dot(num_ref[...], w[:NUM, :], preferred_element_type=jnp.float32) + c
    base_ref[...] = base[:, 0]
    wtab_ref[...] = w[NUM:, 0].reshape(NCAT, EDIM)


def _tc_prep(numerical, W1, b1, W2, b2):
    return pl.pallas_call(
        _tc_prep_body,
        out_shape=[
            jax.ShapeDtypeStruct((B,), jnp.float32),
            jax.ShapeDtypeStruct((NCAT, EDIM), jnp.float32),
        ],
    )(numerical, W1, b1, W2, b2)


def _tc_project_body(t2_ref, wtab_ref, s_ref):
    j = pl.program_id(0)
    mat = t2_ref[...].reshape(EDIM, VOCAB)
    w_row = wtab_ref[pl.ds(j, 1), :]
    vals = jnp.dot(w_row, mat, preferred_element_type=jnp.float32)
    pad = jnp.zeros((1, VP - VOCAB), jnp.float32)
    s_ref[...] = jnp.concatenate([vals, pad], axis=1).reshape(VP)


def _tc_project(t2, wtab):
    return pl.pallas_call(
        _tc_project_body,
        grid=(NCAT,),
        in_specs=[
            pl.BlockSpec((1, EDIM, VOCAB), lambda j: (j, 0, 0)),
            pl.BlockSpec((NCAT, EDIM), lambda j: (0, 0)),
        ],
        out_specs=pl.BlockSpec((VP,), lambda j: (j,)),
        out_shape=jax.ShapeDtypeStruct((NCAT * VP,), jnp.float32),
    )(t2, wtab)


def _tree_sum(terms):
    ts = list(terms)
    while len(ts) > 1:
        nxt = [ts[i] + ts[i + 1] for i in range(0, len(ts) - 1, 2)]
        if len(ts) % 2:
            nxt.append(ts[-1])
        ts = nxt
    return ts[0]


def _sc_body(s_ref, fidx_ref, base_ref, out_ref, idx_v, sbuf, bbuf, obuf, gsem):
    wid = lax.axis_index("s") * 2 + lax.axis_index("c")
    row0 = wid * ROWS_PER_W

    pltpu.sync_copy(fidx_ref.at[wid], idx_v)
    pltpu.sync_copy(base_ref.at[pl.ds(row0, ROWS_PER_W)], bbuf)

    cps = [
        pltpu.async_copy(
            s_ref.at[idx_v.at[j, pl.ds(c * CHUNK, CHUNK)]],
            sbuf.at[j, pl.ds(c * CHUNK, CHUNK)],
            gsem,
        )
        for j in range(NCAT)
        for c in range(CHUNKS)
    ]
    for cp in cps:
        cp.wait()

    def blk_body(k, _):
        off = k * 16
        res = _tree_sum([sbuf[j, pl.ds(off, 16)] for j in range(NCAT)])
        obuf[pl.ds(off, 16)] = res + bbuf[pl.ds(off, 16)]
        return 0

    lax.fori_loop(0, ROWS_PER_W // 16, blk_body, 0)
    pltpu.sync_copy(obuf, out_ref.at[pl.ds(row0, ROWS_PER_W)])


def _sc_gather(s1, fidx, base):
    mesh = plsc.VectorSubcoreMesh(core_axis_name="c", subcore_axis_name="s")
    kfn = pl.kernel(
        _sc_body,
        out_type=jax.ShapeDtypeStruct((B,), jnp.float32),
        mesh=mesh,
        compiler_params=pltpu.CompilerParams(use_tc_tiling_on_sc=False),
        scratch_types=[
            pltpu.VMEM((NCAT, ROWS_PER_W), jnp.int32),
            pltpu.VMEM((NCAT, ROWS_PER_W), jnp.float32),
            pltpu.VMEM((ROWS_PER_W,), jnp.float32),
            pltpu.VMEM((ROWS_PER_W,), jnp.float32),
            pltpu.SemaphoreType.DMA,
        ],
    )
    return kfn(s1, fidx, base)


@jax.jit
def kernel(numerical, cat_indices, emb_tables, W1, b1, W2, b2):
    base, wtab = _tc_prep(
        numerical, W1, b1.reshape(1, HID), W2, b2.reshape(1, NOUT)
    )
    t2 = jnp.transpose(emb_tables, (0, 2, 1))
    s1 = _tc_project(t2, wtab)
    idx3 = (
        cat_indices.astype(jnp.int32)
        .reshape(NW, ROWS_PER_W, NCAT)
        .transpose(0, 2, 1)
    )
    fidx = idx3 + (jnp.arange(NCAT, dtype=jnp.int32) * VP)[None, :, None]
    out = _sc_gather(s1, fidx, base)
    return out.reshape(B, NOUT)


# speedup vs baseline: 21.6787x; 1.0139x over previous
import jax
import jax.numpy as jnp
from jax import lax
from jax.experimental import pallas as pl
from jax.experimental.pallas import tpu as pltpu
from jax.experimental.pallas import tpu_sc as plsc

B = 16384
NUM = 13
NCAT = 26
VOCAB = 100000
EDIM = 16
HID = 16
NOUT = 1

VP = 100352
NW = 32
ROWS_PER_W = B // NW
CHUNK = 128
CHUNKS = ROWS_PER_W // CHUNK


def _tc_prep_body(num_ref, w1_ref, b1_ref, w2_ref, b2_ref, base_ref, wtab_ref):
    w = jnp.dot(w1_ref[...], w2_ref[...], preferred_element_type=jnp.float32)
    c = jnp.dot(b1_ref[...], w2_ref[...], preferred_element_type=jnp.float32) + b2_ref[...]
    base = jnp.dot(num_ref[...], w[:NUM, :], preferred_element_type=jnp.float32) + c
    base_ref[...] = base[:, 0]
    wtab_ref[...] = w[NUM:, 0].reshape(NCAT, EDIM)


def _tc_prep(numerical, W1, b1, W2, b2):
    return pl.pallas_call(
        _tc_prep_body,
        out_shape=[
            jax.ShapeDtypeStruct((B,), jnp.float32),
            jax.ShapeDtypeStruct((NCAT, EDIM), jnp.float32),
        ],
    )(numerical, W1, b1, W2, b2)


NH = NCAT // 2


def _make_tc_project_body(half):
    def body(t2_ref, wtab_ref, s_ref):
        j = pl.program_id(0) + half * NH
        mat = t2_ref[...].reshape(EDIM, VOCAB)
        w_row = wtab_ref[pl.ds(j, 1), :]
        vals = jnp.dot(w_row, mat, preferred_element_type=jnp.float32)
        pad = jnp.zeros((1, VP - VOCAB), jnp.float32)
        s_ref[...] = jnp.concatenate([vals, pad], axis=1).reshape(VP)

    return body


def _tc_project(t2, wtab, half):
    return pl.pallas_call(
        _make_tc_project_body(half),
        grid=(NH,),
        in_specs=[
            pl.BlockSpec((1, EDIM, VOCAB), lambda j: (j + half * NH, 0, 0)),
            pl.BlockSpec((NCAT, EDIM), lambda j: (0, 0)),
        ],
        out_specs=pl.BlockSpec((VP,), lambda j: (j,)),
        out_shape=jax.ShapeDtypeStruct((NH * VP,), jnp.float32),
    )(t2, wtab)


def _tree_sum(terms):
    ts = list(terms)
    while len(ts) > 1:
        nxt = [ts[i] + ts[i + 1] for i in range(0, len(ts) - 1, 2)]
        if len(ts) % 2:
            nxt.append(ts[-1])
        ts = nxt
    return ts[0]


def _sc_body(s_ref, fidx_ref, base_ref, out_ref, idx_v, sbuf, bbuf, obuf, gsem):
    wid = lax.axis_index("s") * 2 + lax.axis_index("c")
    row0 = wid * ROWS_PER_W

    pltpu.sync_copy(fidx_ref.at[wid], idx_v)
    pltpu.sync_copy(base_ref.at[pl.ds(row0, ROWS_PER_W)], bbuf)

    cps = [
        pltpu.async_copy(
            s_ref.at[idx_v.at[j, pl.ds(c * CHUNK, CHUNK)]],
            sbuf.at[j, pl.ds(c * CHUNK, CHUNK)],
            gsem,
        )
        for j in range(NH)
        for c in range(CHUNKS)
    ]
    for cp in cps:
        cp.wait()

    def blk_body(k, _):
        off = k * 16
        res = _tree_sum([sbuf[j, pl.ds(off, 16)] for j in range(NH)])
        obuf[pl.ds(off, 16)] = res + bbuf[pl.ds(off, 16)]
        return 0

    lax.fori_loop(0, ROWS_PER_W // 16, blk_body, 0)
    pltpu.sync_copy(obuf, out_ref.at[pl.ds(row0, ROWS_PER_W)])


def _sc_gather(s1, fidx, base):
    mesh = plsc.VectorSubcoreMesh(core_axis_name="c", subcore_axis_name="s")
    kfn = pl.kernel(
        _sc_body,
        out_type=jax.ShapeDtypeStruct((B,), jnp.float32),
        mesh=mesh,
        compiler_params=pltpu.CompilerParams(use_tc_tiling_on_sc=False),
        scratch_types=[
            pltpu.VMEM((NH, ROWS_PER_W), jnp.int32),
            pltpu.VMEM((NH, ROWS_PER_W), jnp.float32),
            pltpu.VMEM((ROWS_PER_W,), jnp.float32),
            pltpu.VMEM((ROWS_PER_W,), jnp.float32),
            pltpu.SemaphoreType.DMA,
        ],
    )
    return kfn(s1, fidx, base)


@jax.jit
def kernel(numerical, cat_indices, emb_tables, W1, b1, W2, b2):
    base, wtab = _tc_prep(
        numerical, W1, b1.reshape(1, HID), W2, b2.reshape(1, NOUT)
    )
    t2 = jnp.transpose(emb_tables, (0, 2, 1))
    idx3 = (
        cat_indices.astype(jnp.int32)
        .reshape(NW, ROWS_PER_W, NCAT)
        .transpose(0, 2, 1)
    )
    offs = (jnp.arange(NH, dtype=jnp.int32) * VP)[None, :, None]
    fidx_a = idx3[:, :NH, :] + offs
    fidx_b = idx3[:, NH:, :] + offs
    s_a = _tc_project(t2, wtab, 0)
    s_b = _tc_project(t2, wtab, 1)
    part = _sc_gather(s_a, fidx_a, base)
    out = _sc_gather(s_b, fidx_b, part)
    return out.reshape(B, NOUT)


# speedup vs baseline: 22.0361x; 1.0165x over previous
import jax
import jax.numpy as jnp
from jax import lax
from jax.experimental import pallas as pl
from jax.experimental.pallas import tpu as pltpu
from jax.experimental.pallas import tpu_sc as plsc

B = 16384
NUM = 13
NCAT = 26
VOCAB = 100000
EDIM = 16
HID = 16
NOUT = 1

VP = 100352
NW = 32
ROWS_PER_W = B // NW
CHUNK = 128
CHUNKS = ROWS_PER_W // CHUNK


def _tc_prep_body(num_ref, w1_ref, b1_ref, w2_ref, b2_ref, base_ref, wtab_ref):
    w = jnp.dot(w1_ref[...], w2_ref[...], preferred_element_type=jnp.float32)
    c = jnp.dot(b1_ref[...], w2_ref[...], preferred_element_type=jnp.float32) + b2_ref[...]
    base = jnp.dot(num_ref[...], w[:NUM, :], preferred_element_type=jnp.float32) + c
    base_ref[...] = base[:, 0]
    wtab_ref[...] = w[NUM:, 0].reshape(NCAT, EDIM)


def _tc_prep(numerical, W1, b1, W2, b2):
    return pl.pallas_call(
        _tc_prep_body,
        out_shape=[
            jax.ShapeDtypeStruct((B,), jnp.float32),
            jax.ShapeDtypeStruct((NCAT, EDIM), jnp.float32),
        ],
    )(numerical, W1, b1, W2, b2)


NH = NCAT // 2


def _make_tc_project_body(half):
    def body(t2_ref, wtab_ref, s_ref):
        j = pl.program_id(0) + half * NH
        mat = t2_ref[...].reshape(EDIM, VOCAB)
        w_row = wtab_ref[pl.ds(j, 1), :]
        vals = jnp.dot(w_row, mat, preferred_element_type=jnp.float32)
        pad = jnp.zeros((1, VP - VOCAB), jnp.float32)
        s_ref[...] = jnp.concatenate([vals, pad], axis=1).reshape(VP)

    return body


def _tc_project(t2, wtab, half):
    return pl.pallas_call(
        _make_tc_project_body(half),
        grid=(NH,),
        in_specs=[
            pl.BlockSpec((1, EDIM, VOCAB), lambda j: (j + half * NH, 0, 0)),
            pl.BlockSpec((NCAT, EDIM), lambda j: (0, 0)),
        ],
        out_specs=pl.BlockSpec((VP,), lambda j: (j,)),
        out_shape=jax.ShapeDtypeStruct((NH * VP,), jnp.float32),
    )(t2, wtab)


def _tree_sum(terms):
    ts = list(terms)
    while len(ts) > 1:
        nxt = [ts[i] + ts[i + 1] for i in range(0, len(ts) - 1, 2)]
        if len(ts) % 2:
            nxt.append(ts[-1])
        ts = nxt
    return ts[0]


def _make_sc_body(half):
    def body(s_ref, idx_ref, base_ref, out_ref, idx_v, idx_h, sbuf, bbuf, obuf, gsem):
        wid = lax.axis_index("s") * 2 + lax.axis_index("c")
        row0 = wid * ROWS_PER_W

        pltpu.sync_copy(idx_ref.at[:, pl.ds(row0, ROWS_PER_W)], idx_v)
        pltpu.sync_copy(base_ref.at[pl.ds(row0, ROWS_PER_W)], bbuf)

        def add_body(k, _):
            off = k * 16
            for j in range(NH):
                idx_h[j, pl.ds(off, 16)] = (
                    idx_v[half * NH + j, pl.ds(off, 16)] + j * VP
                )
            return 0

        lax.fori_loop(0, ROWS_PER_W // 16, add_body, 0)

        cps = [
            pltpu.async_copy(
                s_ref.at[idx_h.at[j, pl.ds(c * CHUNK, CHUNK)]],
                sbuf.at[j, pl.ds(c * CHUNK, CHUNK)],
                gsem,
            )
            for j in range(NH)
            for c in range(CHUNKS)
        ]
        for cp in cps:
            cp.wait()

        def blk_body(k, _):
            off = k * 16
            res = _tree_sum([sbuf[j, pl.ds(off, 16)] for j in range(NH)])
            obuf[pl.ds(off, 16)] = res + bbuf[pl.ds(off, 16)]
            return 0

        lax.fori_loop(0, ROWS_PER_W // 16, blk_body, 0)
        pltpu.sync_copy(obuf, out_ref.at[pl.ds(row0, ROWS_PER_W)])

    return body


def _sc_gather(s1, idxT, base, half):
    mesh = plsc.VectorSubcoreMesh(core_axis_name="c", subcore_axis_name="s")
    kfn = pl.kernel(
        _make_sc_body(half),
        out_type=jax.ShapeDtypeStruct((B,), jnp.float32),
        mesh=mesh,
        compiler_params=pltpu.CompilerParams(use_tc_tiling_on_sc=False),
        scratch_types=[
            pltpu.VMEM((NCAT, ROWS_PER_W), jnp.int32),
            pltpu.VMEM((NH, ROWS_PER_W), jnp.int32),
            pltpu.VMEM((NH, ROWS_PER_W), jnp.float32),
            pltpu.VMEM((ROWS_PER_W,), jnp.float32),
            pltpu.VMEM((ROWS_PER_W,), jnp.float32),
            pltpu.SemaphoreType.DMA,
        ],
    )
    return kfn(s1, idxT, base)


@jax.jit
def kernel(numerical, cat_indices, emb_tables, W1, b1, W2, b2):
    base, wtab = _tc_prep(
        numerical, W1, b1.reshape(1, HID), W2, b2.reshape(1, NOUT)
    )
    t2 = jnp.transpose(emb_tables, (0, 2, 1))
    idxT = jnp.transpose(cat_indices.astype(jnp.int32))
    s_a = _tc_project(t2, wtab, 0)
    s_b = _tc_project(t2, wtab, 1)
    part = _sc_gather(s_a, idxT, base, 0)
    out = _sc_gather(s_b, idxT, part, 1)
    return out.reshape(B, NOUT)
